# Initial kernel scaffold; baseline (speedup 1.0000x reference)
import jax
import jax.numpy as jnp
from jax.experimental import pallas as pl


def kernel(x, pos_embedding):
    raise NotImplementedError("write your pallas kernel here")


# speedup vs baseline: 1.7177x; 1.7177x over previous
import jax
import jax.numpy as jnp
from jax.experimental import pallas as pl


_TS = 256


def _body(x_ref, p_ref, o_ref):
    o_ref[...] = x_ref[...] + p_ref[...][None, :, :]


def kernel(x, pos_embedding):
    B, S, D = x.shape
    ts = _TS if S % _TS == 0 else S
    grid = (S // ts,)
    return pl.pallas_call(
        _body,
        grid=grid,
        in_specs=[
            pl.BlockSpec((B, ts, D), lambda i: (0, i, 0)),
            pl.BlockSpec((ts, D), lambda i: (i, 0)),
        ],
        out_specs=pl.BlockSpec((B, ts, D), lambda i: (0, i, 0)),
        out_shape=jax.ShapeDtypeStruct((B, S, D), x.dtype),
    )(x, pos_embedding[:S])
